# baseline (speedup 1.0000x reference)
import functools

import jax
import jax.numpy as jnp
from jax import lax
from jax.experimental import pallas as pl
from jax.experimental.pallas import tpu as pltpu
from jax.experimental.pallas import tpu_sc as plsc

NUM_CORES = 2
NUM_SUBCORES = 16
NUM_WORKERS = NUM_CORES * NUM_SUBCORES
LANES = 16

BATCH = 16384
DIM = 64
B_PER_W = BATCH // NUM_WORKERS
IDX_CHUNK = 128
N_CHUNKS = B_PER_W // IDX_CHUNK


def _body(e1_ref, r_ref, ent_ref, rel_ref, out_ref,
          eidx, ridx, erows, rrows, esem, rsem):
    wid = lax.axis_index("s") * NUM_CORES + lax.axis_index("c")
    base = wid * B_PER_W

    pltpu.sync_copy(e1_ref.at[pl.ds(wid * N_CHUNKS, N_CHUNKS)], eidx)
    pltpu.sync_copy(r_ref.at[pl.ds(wid * N_CHUNKS, N_CHUNKS)], ridx)

    copies = []
    for k in range(N_CHUNKS):
        dst = erows.at[pl.ds(k * IDX_CHUNK, IDX_CHUNK)]
        copies.append(pltpu.async_copy(ent_ref.at[eidx.at[k]], dst, esem))
        dstr = rrows.at[pl.ds(k * IDX_CHUNK, IDX_CHUNK)]
        copies.append(pltpu.async_copy(rel_ref.at[ridx.at[k]], dstr, rsem))
    for c in copies:
        c.wait()

    def add_row(i, carry):
        for j in range(DIM // LANES):
            sl = pl.ds(j * LANES, LANES)
            erows[i, sl] = erows[i, sl] + rrows[i, sl]
        return carry

    lax.fori_loop(0, B_PER_W, add_row, 0)

    pltpu.sync_copy(erows, out_ref.at[pl.ds(base, B_PER_W)])


@jax.jit
def _transe(e1_2d, r_2d, entity_table, relation_table):
    mesh = plsc.VectorSubcoreMesh(core_axis_name="c", subcore_axis_name="s")
    kern = pl.kernel(
        _body,
        mesh=mesh,
        compiler_params=pltpu.CompilerParams(use_tc_tiling_on_sc=False),
        out_type=jax.ShapeDtypeStruct((BATCH, DIM), jnp.float32),
        scratch_types=[
            pltpu.VMEM((N_CHUNKS, IDX_CHUNK), jnp.int32),
            pltpu.VMEM((N_CHUNKS, IDX_CHUNK), jnp.int32),
            pltpu.VMEM((B_PER_W, DIM), jnp.float32),
            pltpu.VMEM((B_PER_W, DIM), jnp.float32),
            pltpu.SemaphoreType.DMA,
            pltpu.SemaphoreType.DMA,
        ],
    )
    return kern(e1_2d, r_2d, entity_table, relation_table)


def kernel(e1, r, entity_table, relation_table):
    e1_2d = e1.reshape(BATCH // IDX_CHUNK, IDX_CHUNK)
    r_2d = r.reshape(BATCH // IDX_CHUNK, IDX_CHUNK)
    return _transe(e1_2d, r_2d, entity_table, relation_table)
